# baseline (speedup 1.0000x reference)
import functools

import jax
import jax.numpy as jnp
from jax import lax
from jax.experimental import pallas as pl
from jax.experimental.pallas import tpu as pltpu
from jax.experimental.pallas import tpu_sc as plsc

NC, NS = 2, 16
NW = NC * NS
CHUNK = 128
LANES = 16


def kernel(input, weight):
    B0, B1 = input.shape
    D = weight.shape[1]
    N = B0 * B1
    n_chunks = N // CHUNK
    cpw = n_chunks // NW
    D2 = 2 * D

    idx = input.reshape(n_chunks, CHUNK).astype(jnp.int32)
    slabs = weight.reshape(weight.shape[0] // 2, D2)

    mesh = plsc.VectorSubcoreMesh(core_axis_name="c", subcore_axis_name="s")

    @functools.partial(
        pl.kernel,
        mesh=mesh,
        out_type=jax.ShapeDtypeStruct((N * D,), jnp.float32),
        scratch_types=[
            pltpu.VMEM((cpw, CHUNK), jnp.int32),
            pltpu.VMEM((CHUNK,), jnp.int32),
            pltpu.VMEM((CHUNK, D2), jnp.float32),
            pltpu.VMEM((CHUNK * D,), jnp.float32),
            pltpu.SemaphoreType.DMA,
        ],
    )
    def sc_gather(table_hbm, idx_hbm, out_hbm, idx_v, slab_v, rows_v, out_v,
                  sem):
        wid = lax.axis_index("s") * NC + lax.axis_index("c")
        crow = wid * cpw
        pltpu.sync_copy(idx_hbm.at[pl.ds(crow, cpw)], idx_v)

        @pl.loop(0, cpw)
        def _(j):
            for b in range(CHUNK // LANES):
                sl = pl.ds(b * LANES, LANES)
                slab_v[sl] = jax.lax.shift_right_logical(idx_v[j, sl], 1)
            pltpu.async_copy(table_hbm.at[slab_v], rows_v, sem).wait()

            @pl.loop(0, CHUNK // LANES)
            def _(g):
                offv = (idx_v[j, pl.ds(g * LANES, LANES)] & 1) * D
                for r16 in range(LANES):
                    r = g * LANES + r16
                    off = offv[r16]
                    for k in range(D // LANES):
                        out_v[pl.ds(r * D + k * LANES, LANES)] = (
                            rows_v[r, pl.ds(off + k * LANES, LANES)])

            pltpu.sync_copy(
                out_v, out_hbm.at[pl.ds((crow + j) * CHUNK * D, CHUNK * D)])

    flat = sc_gather(slabs, idx)
    return flat.reshape(B0, B1, D)


# speedup vs baseline: 1.0007x; 1.0007x over previous
import functools

import jax
import jax.numpy as jnp
from jax import lax
from jax.experimental import pallas as pl
from jax.experimental.pallas import tpu as pltpu
from jax.experimental.pallas import tpu_sc as plsc

NC, NS = 2, 16
NW = NC * NS
CHUNK = 128
LANES = 16
FOLD_BLOCK = 4000


def _fold_body(top_ref, bot_ref, o_ref):
    o_ref[:, :top_ref.shape[1]] = top_ref[...]
    o_ref[:, top_ref.shape[1]:] = bot_ref[...]


def kernel(input, weight):
    B0, B1 = input.shape
    V, D = weight.shape
    H = V // 2
    D2 = 2 * D
    N = B0 * B1
    n_chunks = N // CHUNK
    cpw = n_chunks // NW

    idx = input.reshape(n_chunks, CHUNK).astype(jnp.int32)

    slabs = pl.pallas_call(
        _fold_body,
        grid=(H // FOLD_BLOCK,),
        in_specs=[
            pl.BlockSpec((FOLD_BLOCK, D), lambda i: (i, 0)),
            pl.BlockSpec((FOLD_BLOCK, D), lambda i: (i + H // FOLD_BLOCK, 0)),
        ],
        out_specs=pl.BlockSpec((FOLD_BLOCK, D2), lambda i: (i, 0)),
        out_shape=jax.ShapeDtypeStruct((H, D2), jnp.float32),
    )(weight, weight)

    mesh = plsc.VectorSubcoreMesh(core_axis_name="c", subcore_axis_name="s")

    @functools.partial(
        pl.kernel,
        mesh=mesh,
        out_type=jax.ShapeDtypeStruct((N * D,), jnp.float32),
        scratch_types=[
            pltpu.VMEM((cpw, CHUNK), jnp.int32),
            pltpu.VMEM((CHUNK,), jnp.int32),
            pltpu.VMEM((CHUNK, D2), jnp.float32),
            pltpu.VMEM((CHUNK * D,), jnp.float32),
            pltpu.SemaphoreType.DMA,
        ],
    )
    def sc_gather(table_hbm, idx_hbm, out_hbm, idx_v, slab_v, rows_v, out_v,
                  sem):
        wid = lax.axis_index("s") * NC + lax.axis_index("c")
        crow = wid * cpw
        pltpu.sync_copy(idx_hbm.at[pl.ds(crow, cpw)], idx_v)

        @pl.loop(0, cpw)
        def _(j):
            for b in range(CHUNK // LANES):
                sl = pl.ds(b * LANES, LANES)
                iv = idx_v[j, sl]
                slab_v[sl] = jnp.where(iv >= H, iv - H, iv)
            pltpu.async_copy(table_hbm.at[slab_v], rows_v, sem).wait()

            @pl.loop(0, CHUNK // LANES)
            def _(g):
                offv = jnp.where(idx_v[j, pl.ds(g * LANES, LANES)] >= H, D, 0)
                for r16 in range(LANES):
                    r = g * LANES + r16
                    off = offv[r16]
                    for k in range(D // LANES):
                        out_v[pl.ds(r * D + k * LANES, LANES)] = (
                            rows_v[r, pl.ds(off + k * LANES, LANES)])

            pltpu.sync_copy(
                out_v, out_hbm.at[pl.ds((crow + j) * CHUNK * D, CHUNK * D)])

    flat = sc_gather(slabs, idx)
    return flat.reshape(B0, B1, D)


# speedup vs baseline: 1.1203x; 1.1195x over previous
import functools

import jax
import jax.numpy as jnp
from jax import lax
from jax.experimental import pallas as pl
from jax.experimental.pallas import tpu as pltpu
from jax.experimental.pallas import tpu_sc as plsc

NC, NS = 2, 16
NW = NC * NS
CHUNK = 128
LANES = 16
FOLD_BLOCK = 4000


def _fold_body(x_ref, o_ref):
    d = x_ref.shape[2]
    o_ref[:, :d] = x_ref[0]
    o_ref[:, d:] = x_ref[1]


def kernel(input, weight):
    B0, B1 = input.shape
    V, D = weight.shape
    H = V // 2
    D2 = 2 * D
    N = B0 * B1
    n_chunks = N // CHUNK
    cpw = n_chunks // NW

    idx = input.reshape(n_chunks, CHUNK).astype(jnp.int32)

    slabs = pl.pallas_call(
        _fold_body,
        grid=(H // FOLD_BLOCK,),
        in_specs=[
            pl.BlockSpec((2, FOLD_BLOCK, D), lambda i: (0, i, 0)),
        ],
        out_specs=pl.BlockSpec((FOLD_BLOCK, D2), lambda i: (i, 0)),
        out_shape=jax.ShapeDtypeStruct((H, D2), jnp.float32),
    )(weight.reshape(2, H, D))

    mesh = plsc.VectorSubcoreMesh(core_axis_name="c", subcore_axis_name="s")

    @functools.partial(
        pl.kernel,
        mesh=mesh,
        out_type=jax.ShapeDtypeStruct((N * D,), jnp.float32),
        scratch_types=[
            pltpu.VMEM((cpw, CHUNK), jnp.int32),
            pltpu.VMEM((CHUNK,), jnp.int32),
            pltpu.VMEM((CHUNK, D2), jnp.float32),
            pltpu.VMEM((CHUNK * D,), jnp.float32),
            pltpu.SemaphoreType.DMA,
        ],
    )
    def sc_gather(table_hbm, idx_hbm, out_hbm, idx_v, slab_v, rows_v, out_v,
                  sem):
        wid = lax.axis_index("s") * NC + lax.axis_index("c")
        crow = wid * cpw
        pltpu.sync_copy(idx_hbm.at[pl.ds(crow, cpw)], idx_v)

        @pl.loop(0, cpw)
        def _(j):
            for b in range(CHUNK // LANES):
                sl = pl.ds(b * LANES, LANES)
                iv = idx_v[j, sl]
                slab_v[sl] = jnp.where(iv >= H, iv - H, iv)
            pltpu.async_copy(table_hbm.at[slab_v], rows_v, sem).wait()

            @pl.loop(0, CHUNK // LANES)
            def _(g):
                offv = jnp.where(idx_v[j, pl.ds(g * LANES, LANES)] >= H, D, 0)
                for r16 in range(LANES):
                    r = g * LANES + r16
                    off = offv[r16]
                    for k in range(D // LANES):
                        out_v[pl.ds(r * D + k * LANES, LANES)] = (
                            rows_v[r, pl.ds(off + k * LANES, LANES)])

            pltpu.sync_copy(
                out_v, out_hbm.at[pl.ds((crow + j) * CHUNK * D, CHUNK * D)])

    flat = sc_gather(slabs, idx)
    return flat.reshape(B0, B1, D)


# speedup vs baseline: 1.2545x; 1.1198x over previous
import functools

import jax
import jax.numpy as jnp
from jax import lax
from jax.experimental import pallas as pl
from jax.experimental.pallas import tpu as pltpu
from jax.experimental.pallas import tpu_sc as plsc

NC, NS = 2, 16
NW = NC * NS
LANES = 16
FOLD_BLOCK = 4000


def _fold_body(x_ref, o_ref):
    d = x_ref.shape[2]
    o_ref[:, :d] = x_ref[0]
    o_ref[:, d:] = x_ref[1]


def kernel(input, weight):
    B0, B1 = input.shape
    V, D = weight.shape
    H = V // 2
    D2 = 2 * D
    rpw = B0 // NW

    idx = input.astype(jnp.int32)

    slabs = pl.pallas_call(
        _fold_body,
        grid=(H // FOLD_BLOCK,),
        in_specs=[
            pl.BlockSpec((2, FOLD_BLOCK, D), lambda i: (0, i, 0)),
        ],
        out_specs=pl.BlockSpec((FOLD_BLOCK, D2), lambda i: (i, 0)),
        out_shape=jax.ShapeDtypeStruct((H, D2), jnp.float32),
    )(weight.reshape(2, H, D))

    mesh = plsc.VectorSubcoreMesh(core_axis_name="c", subcore_axis_name="s")

    @functools.partial(
        pl.kernel,
        mesh=mesh,
        out_type=jax.ShapeDtypeStruct((B0, B1, D), jnp.float32),
        scratch_types=[
            pltpu.VMEM((rpw, B1), jnp.int32),
            pltpu.VMEM((2, B1), jnp.int32),
            pltpu.VMEM((2, B1, D2), jnp.float32),
            pltpu.VMEM((2, B1, D), jnp.float32),
            pltpu.SemaphoreType.DMA,
            pltpu.SemaphoreType.DMA,
            pltpu.SemaphoreType.DMA,
            pltpu.SemaphoreType.DMA,
        ],
    )
    def sc_gather(table_hbm, idx_hbm, out_hbm, idx_v, slab_v, rows_v, out_v,
                  g0, g1, w0, w1):
        wid = lax.axis_index("s") * NC + lax.axis_index("c")
        base = wid * rpw
        pltpu.sync_copy(idx_hbm.at[pl.ds(base, rpw)], idx_v)

        gsems = (g0, g1)
        wsems = (w0, w1)
        groups = ((0, 0), (B1 - LANES, LANES - (B1 - LANES)))

        def compute_slabs(t, p):
            for gstart, _ in groups:
                sl = pl.ds(gstart, LANES)
                iv = idx_v[t, sl]
                slab_v[p, sl] = jnp.where(iv >= H, iv - H, iv)

        def start_gather(t, p):
            pltpu.async_copy(table_hbm.at[slab_v.at[p]], rows_v.at[p],
                             gsems[p])

        def wait_gather(p):
            pltpu.make_async_copy(table_hbm.at[slab_v.at[p]], rows_v.at[p],
                                  gsems[p]).wait()

        def start_write(t, p):
            pltpu.async_copy(out_v.at[p], out_hbm.at[base + t], wsems[p])

        def wait_write(p):
            pltpu.make_async_copy(out_v.at[p], out_hbm.at[0], wsems[p]).wait()

        def select(t, p):
            for gstart, lane0 in groups:
                offv = jnp.where(idx_v[t, pl.ds(gstart, LANES)] >= H, D, 0)
                for lane in range(lane0, LANES):
                    r = gstart + lane
                    off = offv[lane]
                    for k in range(D // LANES):
                        out_v[p, r, pl.ds(k * LANES, LANES)] = (
                            rows_v[p, r, pl.ds(off + k * LANES, LANES)])

        compute_slabs(0, 0)
        start_gather(0, 0)
        compute_slabs(1, 1)
        start_gather(1, 1)

        @pl.loop(0, rpw // 2)
        def _(h):
            for p in range(2):
                t = 2 * h + p
                wait_gather(p)

                @pl.when(h > 0)
                def _():
                    wait_write(p)

                select(t, p)
                start_write(t, p)

                @pl.when(h < rpw // 2 - 1)
                def _():
                    compute_slabs(t + 2, p)
                    start_gather(t + 2, p)

        wait_write(0)
        wait_write(1)

    return sc_gather(slabs, idx)


# speedup vs baseline: 1.3753x; 1.0963x over previous
import functools

import jax
import jax.numpy as jnp
from jax import lax
from jax.experimental import pallas as pl
from jax.experimental.pallas import tpu as pltpu
from jax.experimental.pallas import tpu_sc as plsc

NC, NS = 2, 16
NW = NC * NS
LANES = 16
FOLD_BLOCK = 4000
ROWS_PER_CHUNK = 4


def _fold_body(x_ref, o_ref):
    d = x_ref.shape[2]
    o_ref[:, :d] = x_ref[0]
    o_ref[:, d:] = x_ref[1]


def kernel(input, weight):
    B0, B1 = input.shape
    V, D = weight.shape
    H = V // 2
    D2 = 2 * D
    rpw = B0 // NW
    W = ROWS_PER_CHUNK * B1
    cpw = rpw // ROWS_PER_CHUNK
    n_chunks = B0 // ROWS_PER_CHUNK

    idx = input.reshape(n_chunks, W).astype(jnp.int32)

    slabs = pl.pallas_call(
        _fold_body,
        grid=(H // FOLD_BLOCK,),
        in_specs=[
            pl.BlockSpec((2, FOLD_BLOCK, D), lambda i: (0, i, 0)),
        ],
        out_specs=pl.BlockSpec((FOLD_BLOCK, D2), lambda i: (i, 0)),
        out_shape=jax.ShapeDtypeStruct((H, D2), jnp.float32),
    )(weight.reshape(2, H, D))

    mesh = plsc.VectorSubcoreMesh(core_axis_name="c", subcore_axis_name="s")

    starts = list(range(0, W - LANES + 1, LANES))
    if starts[-1] != W - LANES:
        starts.append(W - LANES)
    prev_end = 0
    groups = []
    for s in starts:
        groups.append((s, prev_end - s))
        prev_end = s + LANES

    @functools.partial(
        pl.kernel,
        mesh=mesh,
        out_type=jax.ShapeDtypeStruct((B0, B1, D), jnp.float32),
        scratch_types=[
            pltpu.VMEM((cpw, W), jnp.int32),
            pltpu.VMEM((2, W), jnp.int32),
            pltpu.VMEM((2, W, D2), jnp.float32),
            pltpu.VMEM((2, ROWS_PER_CHUNK, B1, D), jnp.float32),
            pltpu.SemaphoreType.DMA,
            pltpu.SemaphoreType.DMA,
            pltpu.SemaphoreType.DMA,
            pltpu.SemaphoreType.DMA,
        ],
    )
    def sc_gather(table_hbm, idx_hbm, out_hbm, idx_v, slab_v, rows_v, out_v,
                  g0, g1, w0, w1):
        wid = lax.axis_index("s") * NC + lax.axis_index("c")
        cbase = wid * cpw
        rbase = wid * rpw
        pltpu.sync_copy(idx_hbm.at[pl.ds(cbase, cpw)], idx_v)

        gsems = (g0, g1)
        wsems = (w0, w1)

        def compute_slabs(t, p):
            for gstart, _ in groups:
                sl = pl.ds(gstart, LANES)
                iv = idx_v[t, sl]
                slab_v[p, sl] = jnp.where(iv >= H, iv - H, iv)

        def start_gather(t, p):
            pltpu.async_copy(table_hbm.at[slab_v.at[p]], rows_v.at[p],
                             gsems[p])

        def wait_gather(p):
            pltpu.make_async_copy(table_hbm.at[slab_v.at[p]], rows_v.at[p],
                                  gsems[p]).wait()

        def start_write(t, p):
            pltpu.async_copy(
                out_v.at[p],
                out_hbm.at[pl.ds(rbase + t * ROWS_PER_CHUNK, ROWS_PER_CHUNK)],
                wsems[p])

        def wait_write(p):
            pltpu.make_async_copy(
                out_v.at[p], out_hbm.at[pl.ds(0, ROWS_PER_CHUNK)],
                wsems[p]).wait()

        def select(t, p):
            for gstart, lane0 in groups:
                offv = jnp.where(idx_v[t, pl.ds(gstart, LANES)] >= H, D, 0)
                for lane in range(lane0, LANES):
                    r = gstart + lane
                    q, rr = divmod(r, B1)
                    off = offv[lane]
                    for k in range(D // LANES):
                        out_v[p, q, rr, pl.ds(k * LANES, LANES)] = (
                            rows_v[p, r, pl.ds(off + k * LANES, LANES)])

        compute_slabs(0, 0)
        start_gather(0, 0)
        compute_slabs(1, 1)
        start_gather(1, 1)

        @pl.loop(0, cpw // 2)
        def _(h):
            for p in range(2):
                t = 2 * h + p
                wait_gather(p)

                @pl.when(h > 0)
                def _():
                    wait_write(p)

                select(t, p)
                start_write(t, p)

                @pl.when(h < cpw // 2 - 1)
                def _():
                    compute_slabs(t + 2, p)
                    start_gather(t + 2, p)

        wait_write(0)
        wait_write(1)

    return sc_gather(slabs, idx)
